# baseline (speedup 1.0000x reference)
import functools

import jax
import jax.numpy as jnp
from jax import lax
from jax.experimental import pallas as pl
from jax.experimental.pallas import tpu as pltpu
from jax.experimental.pallas import tpu_sc as plsc

ROWS = 128
COLS = 100000
LANES = 16

_info = plsc.get_sparse_core_info()
_NC, _NS = _info.num_cores, _info.num_subcores
NWORKERS = _NC * _NS
CW = 400
SC_COLS = 38400
NBLOCKS = SC_COLS // CW
FULL_J = NBLOCKS // NWORKERS
NRB = ROWS // LANES

TC_START = 36000
TC_BR = 4000
TC_OFF_B = TC_START // TC_BR
TC_STEPS = (COLS - TC_START) // TC_BR

_NEG_INF = float("-inf")
_BIG_I32 = 0x7FFFFFFF


@functools.partial(
    pl.kernel,
    out_type=(
        jax.ShapeDtypeStruct((NWORKERS, ROWS), jnp.float32),
        jax.ShapeDtypeStruct((NWORKERS, ROWS), jnp.int32),
    ),
    mesh=plsc.VectorSubcoreMesh(core_axis_name="c", subcore_axis_name="s"),
    compiler_params=pltpu.CompilerParams(
        needs_layout_passes=False,
        skip_device_barrier=True,
        disable_bounds_checks=True,
        disable_semaphore_checks=True,
    ),
    scratch_types=[
        pltpu.VMEM((CW, ROWS), jnp.float32),
        pltpu.VMEM((CW, ROWS), jnp.float32),
        pltpu.VMEM((ROWS,), jnp.float32),
        pltpu.VMEM((ROWS,), jnp.int32),
        pltpu.VMEM_SHARED((_NS, ROWS), jnp.float32),
        pltpu.VMEM_SHARED((_NS, ROWS), jnp.int32),
        pltpu.SemaphoreType.DMA,
        pltpu.SemaphoreType.DMA,
    ],
)
def _sc_argmax(xt_hbm, oval_hbm, oidx_hbm, buf0, buf1, vbuf, ibuf,
               sh_val, sh_idx, sem0, sem1):
    cid = lax.axis_index("c")
    sid = lax.axis_index("s")
    wid = cid * _NS + sid
    bufs = (buf0, buf1)
    sems = (sem0, sem1)

    def start(j):
        blk = j * NWORKERS + wid
        off = pl.multiple_of(blk * CW, 8)
        return pltpu.async_copy(
            xt_hbm.at[pl.ds(off, CW), :], bufs[j % 2], sems[j % 2]
        )

    copies = {0: start(0), 1: start(1)}

    m = [jnp.full((LANES,), _NEG_INF, jnp.float32) for _ in range(NRB)]
    a = [jnp.zeros((LANES,), jnp.int32) for _ in range(NRB)]

    for j in range(FULL_J):
        buf = bufs[j % 2]
        col0 = (j * NWORKERS + wid) * CW
        copies[j].wait()

        def body(cc, carry, buf=buf, col0=col0):
            mm = list(carry[0])
            aa = list(carry[1])
            col = col0 + cc
            for rb in range(NRB):
                x = buf[cc, pl.ds(rb * LANES, LANES)]
                gt = x > mm[rb]
                mm[rb] = jnp.where(gt, x, mm[rb])
                aa[rb] = jnp.where(gt, col, aa[rb])
            return tuple(mm), tuple(aa)

        m, a = lax.fori_loop(0, CW, body, (tuple(m), tuple(a)))
        m, a = list(m), list(a)
        if j + 2 < FULL_J:
            copies[j + 2] = start(j + 2)

    for rb in range(NRB):
        vbuf[pl.ds(rb * LANES, LANES)] = m[rb]
        ibuf[pl.ds(rb * LANES, LANES)] = a[rb]
    pltpu.sync_copy(vbuf, sh_val.at[sid])
    pltpu.sync_copy(ibuf, sh_idx.at[sid])
    plsc.subcore_barrier()

    @pl.when(sid == 0)
    def _():
        row0 = pl.multiple_of(cid * _NS, 8)
        pltpu.sync_copy(sh_val, oval_hbm.at[pl.ds(row0, _NS), :])
        pltpu.sync_copy(sh_idx, oidx_hbm.at[pl.ds(row0, _NS), :])


def _make_tc_scan(start, steps):
    off_b = start // TC_BR

    def body(x_ref, val_ref, idx_ref, m_scr, a_scr):
        i = pl.program_id(0)
        bs = start + i * TC_BR

        @pl.when(i == 0)
        def _():
            m_scr[...] = jnp.full((8, ROWS), _NEG_INF, jnp.float32)
            a_scr[...] = jnp.zeros((8, ROWS), jnp.int32)

        x = x_ref[...]
        m = m_scr[...]
        a = a_scr[...]
        for k in range(TC_BR // 8):
            xk = x[k * 8:(k + 1) * 8, :]
            gt = xk > m
            m = jnp.maximum(m, xk)
            a = jnp.where(gt, bs + k * 8, a)
        m_scr[...] = m
        a_scr[...] = a

        @pl.when(i == steps - 1)
        def _():
            srow = jax.lax.broadcasted_iota(jnp.int32, (8, ROWS), 0)
            best = jnp.max(m, axis=0, keepdims=True)
            cand = jnp.where(m == best, a + srow, _BIG_I32)
            val_ref[...] = best
            idx_ref[...] = jnp.min(cand, axis=0, keepdims=True)

    return pl.pallas_call(
        body,
        grid=(steps,),
        in_specs=[pl.BlockSpec((TC_BR, ROWS), lambda i: (off_b + i, 0))],
        out_specs=[
            pl.BlockSpec((1, ROWS), lambda i: (0, 0)),
            pl.BlockSpec((1, ROWS), lambda i: (0, 0)),
        ],
        out_shape=(
            jax.ShapeDtypeStruct((1, ROWS), jnp.float32),
            jax.ShapeDtypeStruct((1, ROWS), jnp.int32),
        ),
        scratch_shapes=[
            pltpu.VMEM((8, ROWS), jnp.float32),
            pltpu.VMEM((8, ROWS), jnp.int32),
        ],
    )


TC_SPLIT = 48000
_tc_scan_a = _make_tc_scan(TC_START, (TC_SPLIT - TC_START) // TC_BR)
_tc_scan_b = _make_tc_scan(TC_SPLIT, (COLS - TC_SPLIT) // TC_BR)


def _merge_body(scv_ref, sci_ref, av_ref, ai_ref, bv_ref, bi_ref, out_ref):
    v = scv_ref[...]
    i = sci_ref[...]
    overall = jnp.maximum(jnp.max(v, axis=0, keepdims=True), av_ref[...])
    overall = jnp.maximum(overall, bv_ref[...])
    cand_sc = jnp.min(
        jnp.where(v == overall, i, _BIG_I32), axis=0, keepdims=True
    )
    cand_a = jnp.where(av_ref[...] == overall, ai_ref[...], _BIG_I32)
    cand_b = jnp.where(bv_ref[...] == overall, bi_ref[...], _BIG_I32)
    out_ref[...] = jnp.minimum(jnp.minimum(cand_sc, cand_a), cand_b)


_merge = pl.pallas_call(
    _merge_body,
    out_shape=jax.ShapeDtypeStruct((1, ROWS), jnp.int32),
)


def kernel(m_logits):
    xt = m_logits.T
    av, ai = _tc_scan_a(xt)
    scv, sci = _sc_argmax(xt)
    bv, bi = _tc_scan_b(xt)
    return _merge(scv, sci, av, ai, bv, bi).reshape(ROWS, 1)


# speedup vs baseline: 1.0376x; 1.0376x over previous
import functools

import jax
import jax.numpy as jnp
from jax import lax
from jax.experimental import pallas as pl
from jax.experimental.pallas import tpu as pltpu
from jax.experimental.pallas import tpu_sc as plsc

ROWS = 128
COLS = 100000
LANES = 16

_info = plsc.get_sparse_core_info()
_NC, _NS = _info.num_cores, _info.num_subcores
NWORKERS = _NC * _NS
CW = 400
SC_COLS = 38400
NBLOCKS = SC_COLS // CW
FULL_J = NBLOCKS // NWORKERS
NRB = ROWS // LANES

TC_START = 36000
TC_BR = 4000
TC_OFF_B = TC_START // TC_BR
TC_STEPS = (COLS - TC_START) // TC_BR

_NEG_INF = float("-inf")
_BIG_I32 = 0x7FFFFFFF


@functools.partial(
    pl.kernel,
    out_type=(
        jax.ShapeDtypeStruct((NWORKERS, ROWS), jnp.float32),
        jax.ShapeDtypeStruct((NWORKERS, ROWS), jnp.int32),
    ),
    mesh=plsc.VectorSubcoreMesh(core_axis_name="c", subcore_axis_name="s"),
    compiler_params=pltpu.CompilerParams(
        needs_layout_passes=False,
        skip_device_barrier=True,
        disable_bounds_checks=True,
        disable_semaphore_checks=True,
    ),
    scratch_types=[
        pltpu.VMEM((CW, ROWS), jnp.float32),
        pltpu.VMEM((CW, ROWS), jnp.float32),
        pltpu.VMEM((ROWS,), jnp.float32),
        pltpu.VMEM((ROWS,), jnp.int32),
        pltpu.VMEM_SHARED((_NS, ROWS), jnp.float32),
        pltpu.VMEM_SHARED((_NS, ROWS), jnp.int32),
        pltpu.SemaphoreType.DMA,
        pltpu.SemaphoreType.DMA,
    ],
)
def _sc_argmax(xt_hbm, oval_hbm, oidx_hbm, buf0, buf1, vbuf, ibuf,
               sh_val, sh_idx, sem0, sem1):
    cid = lax.axis_index("c")
    sid = lax.axis_index("s")
    wid = cid * _NS + sid
    bufs = (buf0, buf1)
    sems = (sem0, sem1)

    def start(j):
        blk = j * NWORKERS + wid
        off = pl.multiple_of(blk * CW, 8)
        return pltpu.async_copy(
            xt_hbm.at[pl.ds(off, CW), :], bufs[j % 2], sems[j % 2]
        )

    copies = {0: start(0), 1: start(1)}

    m = [jnp.full((LANES,), _NEG_INF, jnp.float32) for _ in range(NRB)]
    a = [jnp.zeros((LANES,), jnp.int32) for _ in range(NRB)]

    for j in range(FULL_J):
        buf = bufs[j % 2]
        col0 = (j * NWORKERS + wid) * CW
        copies[j].wait()

        def body(cc, carry, buf=buf, col0=col0):
            mm = list(carry[0])
            aa = list(carry[1])
            col = col0 + cc
            for rb in range(NRB):
                x = buf[cc, pl.ds(rb * LANES, LANES)]
                gt = x > mm[rb]
                mm[rb] = jnp.where(gt, x, mm[rb])
                aa[rb] = jnp.where(gt, col, aa[rb])
            return tuple(mm), tuple(aa)

        m, a = lax.fori_loop(0, CW, body, (tuple(m), tuple(a)))
        m, a = list(m), list(a)
        if j + 2 < FULL_J:
            copies[j + 2] = start(j + 2)

    for rb in range(NRB):
        vbuf[pl.ds(rb * LANES, LANES)] = m[rb]
        ibuf[pl.ds(rb * LANES, LANES)] = a[rb]
    pltpu.sync_copy(vbuf, sh_val.at[sid])
    pltpu.sync_copy(ibuf, sh_idx.at[sid])
    plsc.subcore_barrier()

    @pl.when(sid == 0)
    def _():
        row0 = pl.multiple_of(cid * _NS, 8)
        pltpu.sync_copy(sh_val, oval_hbm.at[pl.ds(row0, _NS), :])
        pltpu.sync_copy(sh_idx, oidx_hbm.at[pl.ds(row0, _NS), :])


TC_HALF = (COLS - TC_START) // 2
TC_STEPS2 = TC_HALF // TC_BR


def _tc_scan_body(x1_ref, x2_ref, val_ref, idx_ref, m1_s, a1_s, m2_s, a2_s):
    i = pl.program_id(0)

    @pl.when(i == 0)
    def _():
        for ms, asc in ((m1_s, a1_s), (m2_s, a2_s)):
            ms[...] = jnp.full((8, ROWS), _NEG_INF, jnp.float32)
            asc[...] = jnp.zeros((8, ROWS), jnp.int32)

    for ref, ms, asc, base in (
        (x1_ref, m1_s, a1_s, TC_START),
        (x2_ref, m2_s, a2_s, TC_START + TC_HALF),
    ):
        bs = base + i * TC_BR
        x = ref[...]
        m = ms[...]
        a = asc[...]
        for k in range(TC_BR // 8):
            xk = x[k * 8:(k + 1) * 8, :]
            gt = xk > m
            m = jnp.maximum(m, xk)
            a = jnp.where(gt, bs + k * 8, a)
        ms[...] = m
        asc[...] = a

    @pl.when(i == TC_STEPS2 - 1)
    def _():
        srow = jax.lax.broadcasted_iota(jnp.int32, (8, ROWS), 0)
        m1 = m1_s[...]
        m2 = m2_s[...]
        gt = m2 > m1
        m = jnp.where(gt, m2, m1)
        a = jnp.where(gt, a2_s[...], a1_s[...])
        best = jnp.max(m, axis=0, keepdims=True)
        cand = jnp.where(m == best, a + srow, _BIG_I32)
        val_ref[...] = best
        idx_ref[...] = jnp.min(cand, axis=0, keepdims=True)


_tc_scan = pl.pallas_call(
    _tc_scan_body,
    grid=(TC_STEPS2,),
    in_specs=[
        pl.BlockSpec((TC_BR, ROWS), lambda i: (TC_START // TC_BR + i, 0)),
        pl.BlockSpec(
            (TC_BR, ROWS), lambda i: ((TC_START + TC_HALF) // TC_BR + i, 0)
        ),
    ],
    out_specs=[
        pl.BlockSpec((1, ROWS), lambda i: (0, 0)),
        pl.BlockSpec((1, ROWS), lambda i: (0, 0)),
    ],
    out_shape=(
        jax.ShapeDtypeStruct((1, ROWS), jnp.float32),
        jax.ShapeDtypeStruct((1, ROWS), jnp.int32),
    ),
    scratch_shapes=[
        pltpu.VMEM((8, ROWS), jnp.float32),
        pltpu.VMEM((8, ROWS), jnp.int32),
        pltpu.VMEM((8, ROWS), jnp.float32),
        pltpu.VMEM((8, ROWS), jnp.int32),
    ],
)


def _merge_body(scv_ref, sci_ref, tcv_ref, tci_ref, out_ref):
    v = scv_ref[...]
    i = sci_ref[...]
    tcv = tcv_ref[...]
    tci = tci_ref[...]
    overall = jnp.maximum(jnp.max(v, axis=0, keepdims=True), tcv)
    cand_sc = jnp.min(
        jnp.where(v == overall, i, _BIG_I32), axis=0, keepdims=True
    )
    cand_tc = jnp.where(tcv == overall, tci, _BIG_I32)
    out_ref[...] = jnp.minimum(cand_sc, cand_tc)


_merge = pl.pallas_call(
    _merge_body,
    out_shape=jax.ShapeDtypeStruct((1, ROWS), jnp.int32),
)


def kernel(m_logits):
    xt = m_logits.T
    scv, sci = _sc_argmax(xt)
    tcv, tci = _tc_scan(xt, xt)
    return _merge(scv, sci, tcv, tci).reshape(ROWS, 1)


# speedup vs baseline: 1.0630x; 1.0245x over previous
import functools

import jax
import jax.numpy as jnp
from jax import lax
from jax.experimental import pallas as pl
from jax.experimental.pallas import tpu as pltpu
from jax.experimental.pallas import tpu_sc as plsc

ROWS = 128
COLS = 100000
LANES = 16

_info = plsc.get_sparse_core_info()
_NC, _NS = _info.num_cores, _info.num_subcores
NWORKERS = _NC * _NS
CW = 200
SC_COLS = 32000
NBLOCKS = SC_COLS // CW
FULL_J = NBLOCKS // NWORKERS
NRB = ROWS // LANES

TC_START = 32000
TC_BR = 2000
TC_OFF_B = TC_START // TC_BR
TC_STEPS = (COLS - TC_START) // TC_BR

_NEG_INF = float("-inf")
_BIG_I32 = 0x7FFFFFFF


@functools.partial(
    pl.kernel,
    out_type=(
        jax.ShapeDtypeStruct((NWORKERS, ROWS), jnp.float32),
        jax.ShapeDtypeStruct((NWORKERS, ROWS), jnp.int32),
    ),
    mesh=plsc.VectorSubcoreMesh(core_axis_name="c", subcore_axis_name="s"),
    compiler_params=pltpu.CompilerParams(
        needs_layout_passes=False,
        skip_device_barrier=True,
        disable_bounds_checks=True,
        disable_semaphore_checks=True,
    ),
    scratch_types=[
        pltpu.VMEM((CW, ROWS), jnp.float32),
        pltpu.VMEM((CW, ROWS), jnp.float32),
        pltpu.VMEM((ROWS,), jnp.float32),
        pltpu.VMEM((ROWS,), jnp.int32),
        pltpu.VMEM_SHARED((_NS, ROWS), jnp.float32),
        pltpu.VMEM_SHARED((_NS, ROWS), jnp.int32),
        pltpu.SemaphoreType.DMA,
        pltpu.SemaphoreType.DMA,
    ],
)
def _sc_argmax(xt_hbm, oval_hbm, oidx_hbm, buf0, buf1, vbuf, ibuf,
               sh_val, sh_idx, sem0, sem1):
    cid = lax.axis_index("c")
    sid = lax.axis_index("s")
    wid = cid * _NS + sid
    bufs = (buf0, buf1)
    sems = (sem0, sem1)

    def start(j):
        blk = j * NWORKERS + wid
        off = pl.multiple_of(blk * CW, 8)
        return pltpu.async_copy(
            xt_hbm.at[pl.ds(off, CW), :], bufs[j % 2], sems[j % 2]
        )

    copies = {0: start(0), 1: start(1)}

    m = [jnp.full((LANES,), _NEG_INF, jnp.float32) for _ in range(NRB)]
    a = [jnp.zeros((LANES,), jnp.int32) for _ in range(NRB)]

    for j in range(FULL_J):
        buf = bufs[j % 2]
        col0 = (j * NWORKERS + wid) * CW
        copies[j].wait()

        def body(cc, carry, buf=buf, col0=col0):
            mm = list(carry[0])
            aa = list(carry[1])
            col = col0 + cc
            for rb in range(NRB):
                x = buf[cc, pl.ds(rb * LANES, LANES)]
                gt = x > mm[rb]
                mm[rb] = jnp.where(gt, x, mm[rb])
                aa[rb] = jnp.where(gt, col, aa[rb])
            return tuple(mm), tuple(aa)

        m, a = lax.fori_loop(0, CW, body, (tuple(m), tuple(a)))
        m, a = list(m), list(a)
        if j + 2 < FULL_J:
            copies[j + 2] = start(j + 2)

    for rb in range(NRB):
        vbuf[pl.ds(rb * LANES, LANES)] = m[rb]
        ibuf[pl.ds(rb * LANES, LANES)] = a[rb]
    pltpu.sync_copy(vbuf, sh_val.at[sid])
    pltpu.sync_copy(ibuf, sh_idx.at[sid])
    plsc.subcore_barrier()

    @pl.when(sid == 0)
    def _():
        row0 = pl.multiple_of(cid * _NS, 8)
        pltpu.sync_copy(sh_val, oval_hbm.at[pl.ds(row0, _NS), :])
        pltpu.sync_copy(sh_idx, oidx_hbm.at[pl.ds(row0, _NS), :])


TC_HALF = (COLS - TC_START) // 2
TC_STEPS2 = TC_HALF // TC_BR


def _tc_scan_body(x1_ref, x2_ref, val_ref, idx_ref, m1_s, a1_s, m2_s, a2_s):
    i = pl.program_id(0)

    @pl.when(i == 0)
    def _():
        for ms, asc in ((m1_s, a1_s), (m2_s, a2_s)):
            ms[...] = jnp.full((8, ROWS), _NEG_INF, jnp.float32)
            asc[...] = jnp.zeros((8, ROWS), jnp.int32)

    for ref, ms, asc, base in (
        (x1_ref, m1_s, a1_s, TC_START),
        (x2_ref, m2_s, a2_s, TC_START + TC_HALF),
    ):
        bs = base + i * TC_BR
        x = ref[...]
        m = ms[...]
        a = asc[...]
        for k in range(TC_BR // 8):
            xk = x[k * 8:(k + 1) * 8, :]
            gt = xk > m
            m = jnp.maximum(m, xk)
            a = jnp.where(gt, bs + k * 8, a)
        ms[...] = m
        asc[...] = a

    @pl.when(i == TC_STEPS2 - 1)
    def _():
        srow = jax.lax.broadcasted_iota(jnp.int32, (8, ROWS), 0)
        m1 = m1_s[...]
        m2 = m2_s[...]
        gt = m2 > m1
        m = jnp.where(gt, m2, m1)
        a = jnp.where(gt, a2_s[...], a1_s[...])
        best = jnp.max(m, axis=0, keepdims=True)
        cand = jnp.where(m == best, a + srow, _BIG_I32)
        val_ref[...] = best
        idx_ref[...] = jnp.min(cand, axis=0, keepdims=True)


_tc_scan = pl.pallas_call(
    _tc_scan_body,
    grid=(TC_STEPS2,),
    in_specs=[
        pl.BlockSpec((TC_BR, ROWS), lambda i: (TC_START // TC_BR + i, 0)),
        pl.BlockSpec(
            (TC_BR, ROWS), lambda i: ((TC_START + TC_HALF) // TC_BR + i, 0)
        ),
    ],
    out_specs=[
        pl.BlockSpec((1, ROWS), lambda i: (0, 0)),
        pl.BlockSpec((1, ROWS), lambda i: (0, 0)),
    ],
    out_shape=(
        jax.ShapeDtypeStruct((1, ROWS), jnp.float32),
        jax.ShapeDtypeStruct((1, ROWS), jnp.int32),
    ),
    scratch_shapes=[
        pltpu.VMEM((8, ROWS), jnp.float32),
        pltpu.VMEM((8, ROWS), jnp.int32),
        pltpu.VMEM((8, ROWS), jnp.float32),
        pltpu.VMEM((8, ROWS), jnp.int32),
    ],
)


def _merge_body(scv_ref, sci_ref, tcv_ref, tci_ref, out_ref):
    v = scv_ref[...]
    i = sci_ref[...]
    tcv = tcv_ref[...]
    tci = tci_ref[...]
    overall = jnp.maximum(jnp.max(v, axis=0, keepdims=True), tcv)
    cand_sc = jnp.min(
        jnp.where(v == overall, i, _BIG_I32), axis=0, keepdims=True
    )
    cand_tc = jnp.where(tcv == overall, tci, _BIG_I32)
    out_ref[...] = jnp.minimum(cand_sc, cand_tc)


_merge = pl.pallas_call(
    _merge_body,
    out_shape=jax.ShapeDtypeStruct((1, ROWS), jnp.int32),
)


def kernel(m_logits):
    xt = m_logits.T
    scv, sci = _sc_argmax(xt)
    tcv, tci = _tc_scan(xt, xt)
    return _merge(scv, sci, tcv, tci).reshape(ROWS, 1)
